# Initial kernel scaffold; baseline (speedup 1.0000x reference)
import jax
import jax.numpy as jnp
from jax import lax
from jax.experimental import pallas as pl
from jax.experimental.pallas import tpu as pltpu
from jax.experimental.pallas import tpu_sc as plsc

N_NODES = 10000
N_EDGES = 320000
D_IN = 128
D_HID = 32
N_CLASSES = 16

_LANES = 16
_NUM_CORES = 2
_E_HALF = N_EDGES // _NUM_CORES
_CHUNK = 10000
_N_CHUNKS = _E_HALF // _CHUNK
_VECS_PER_CHUNK = _CHUNK // _LANES
_N_VECS = N_NODES // _LANES


def _zero_columns(o0, o1):
  zero = jnp.zeros((_LANES,), jnp.float32)

  def step(i, carry):
    sl = pl.ds(i * _LANES, _LANES)
    o0[sl] = zero
    o1[sl] = zero
    return carry

  lax.fori_loop(0, _N_VECS, step, 0)


def _edge_loop(t0, t1, o0, o1, sv, dv, wv):
  def step(i, carry):
    sl = pl.ds(i * _LANES, _LANES)
    s_idx = sv[sl]
    d_idx = dv[sl]
    w = wv[sl]
    g0 = plsc.load_gather(t0, [s_idx]) * w
    g1 = plsc.load_gather(t1, [s_idx]) * w
    plsc.addupdate_scatter(o0, [d_idx], g0)
    plsc.addupdate_scatter(o1, [d_idx], g1)
    return carry

  lax.fori_loop(0, _VECS_PER_CHUNK, step, 0)


def _propagate_chunks(core, src, dst, w, t0, t1, o0, o1, sv, dv, wv):
  ebase = core * _E_HALF
  for k in range(_N_CHUNKS):
    off = ebase + k * _CHUNK
    pltpu.sync_copy(src.at[pl.ds(off, _CHUNK)], sv)
    pltpu.sync_copy(dst.at[pl.ds(off, _CHUNK)], dv)
    pltpu.sync_copy(w.at[pl.ds(off, _CHUNK)], wv)
    _edge_loop(t0, t1, o0, o1, sv, dv, wv)


def _sc_phase1_body(ht, src, dst, w, out, t0, t1, o0, o1, sv, dv, wv, bb):
  c = lax.axis_index("c")
  s = lax.axis_index("s")
  d0 = 2 * s
  d1 = d0 + 1
  pltpu.sync_copy(ht.at[d0], t0)
  pltpu.sync_copy(ht.at[d1], t1)
  _zero_columns(o0, o1)
  _propagate_chunks(c, src, dst, w, t0, t1, o0, o1, sv, dv, wv)
  pltpu.sync_copy(o0, out.at[c, d0])
  pltpu.sync_copy(o1, out.at[c, d1])


def _sc_phase2_body(parts, bias, src, dst, w, out,
                    t0, t1, o0, o1, sv, dv, wv, bb):
  c = lax.axis_index("c")
  s = lax.axis_index("s")
  d0 = 2 * s
  d1 = d0 + 1
  pltpu.sync_copy(parts.at[0, d0], t0)
  pltpu.sync_copy(parts.at[1, d0], o0)
  pltpu.sync_copy(parts.at[0, d1], t1)
  pltpu.sync_copy(parts.at[1, d1], o1)
  pltpu.sync_copy(bias.at[d0], bb)
  b0 = bb[...]
  pltpu.sync_copy(bias.at[d1], bb)
  b1 = bb[...]

  def cstep(i, carry):
    sl = pl.ds(i * _LANES, _LANES)
    t0[sl] = jnp.maximum(t0[sl] + o0[sl] + b0, 0.0)
    t1[sl] = jnp.maximum(t1[sl] + o1[sl] + b1, 0.0)
    return carry

  lax.fori_loop(0, _N_VECS, cstep, 0)

  _zero_columns(o0, o1)
  _propagate_chunks(c, src, dst, w, t0, t1, o0, o1, sv, dv, wv)
  pltpu.sync_copy(o0, out.at[c, d0])
  pltpu.sync_copy(o1, out.at[c, d1])


def _make_sc_kernel(body):
  mesh = plsc.VectorSubcoreMesh(core_axis_name="c", subcore_axis_name="s")
  return pl.kernel(
      body,
      out_type=jax.ShapeDtypeStruct((_NUM_CORES, D_HID, N_NODES),
                                    jnp.float32),
      mesh=mesh,
      scratch_types=[
          pltpu.VMEM((N_NODES,), jnp.float32),
          pltpu.VMEM((N_NODES,), jnp.float32),
          pltpu.VMEM((N_NODES,), jnp.float32),
          pltpu.VMEM((N_NODES,), jnp.float32),
          pltpu.VMEM((_CHUNK,), jnp.int32),
          pltpu.VMEM((_CHUNK,), jnp.int32),
          pltpu.VMEM((_CHUNK,), jnp.float32),
          pltpu.VMEM((_LANES,), jnp.float32),
      ],
  )


def _mm1_body(x_ref, w_ref, out_ref):
  out_ref[...] = lax.dot_general(
      w_ref[...], x_ref[...], (((0,), (1,)), ((), ())),
      preferred_element_type=jnp.float32)


def _final_body(p_ref, w2_ref, b2_ref, out_ref):
  g = p_ref[0] + p_ref[1]
  logits = lax.dot_general(
      g, w2_ref[...], (((0,), (0,)), ((), ())),
      preferred_element_type=jnp.float32)
  z = logits + b2_ref[0][None, :]
  m = jnp.max(z, axis=1, keepdims=True)
  lse = jnp.log(jnp.sum(jnp.exp(z - m), axis=1, keepdims=True)) + m
  out_ref[...] = z - lse


@jax.jit
def kernel(x, edge_index, edge_weight, W1, b1, W2, b2):
  src = edge_index[0].astype(jnp.int32)
  dst = edge_index[1].astype(jnp.int32)
  w = edge_weight.astype(jnp.float32)

  ht = pl.pallas_call(
      _mm1_body,
      out_shape=jax.ShapeDtypeStruct((D_HID, N_NODES), jnp.float32),
  )(x, W1)

  p1 = _make_sc_kernel(_sc_phase1_body)(ht, src, dst, w)

  b1b = jnp.broadcast_to(b1[:, None], (D_HID, _LANES))
  p2 = _make_sc_kernel(_sc_phase2_body)(p1, b1b, src, dst, w)

  out = pl.pallas_call(
      _final_body,
      out_shape=jax.ShapeDtypeStruct((N_NODES, N_CLASSES), jnp.float32),
  )(p2, W2, b2.reshape(1, N_CLASSES))
  return out


# speedup vs baseline: 7.3903x; 7.3903x over previous
import jax
import jax.numpy as jnp
from jax import lax
from jax.experimental import pallas as pl
from jax.experimental.pallas import tpu as pltpu
from jax.experimental.pallas import tpu_sc as plsc

N_NODES = 10000
N_EDGES = 320000
D_IN = 128
D_HID = 32
N_CLASSES = 16

_LANES = 16
_NUM_CORES = 2
_E_HALF = N_EDGES // _NUM_CORES
_CHUNK = 10000
_N_CHUNKS = _E_HALF // _CHUNK
_VECS_PER_CHUNK = _CHUNK // _LANES
_N_VECS = N_NODES // _LANES


def _zero_columns(o0, o1):
  zero = jnp.zeros((_LANES,), jnp.float32)

  def step(i, carry):
    sl = pl.ds(i * _LANES, _LANES)
    o0[sl] = zero
    o1[sl] = zero
    return carry

  lax.fori_loop(0, _N_VECS, step, 0)


def _edge_loop(t0, t1, o0, o1, sv, dv, wv):
  def step(i, carry):
    sl = pl.ds(i * _LANES, _LANES)
    s_idx = sv[sl]
    d_idx = dv[sl]
    w = wv[sl]
    g0 = plsc.load_gather(t0, [s_idx]) * w
    g1 = plsc.load_gather(t1, [s_idx]) * w
    plsc.addupdate_scatter(o0, [d_idx], g0)
    plsc.addupdate_scatter(o1, [d_idx], g1)
    return carry

  lax.fori_loop(0, _VECS_PER_CHUNK, step, 0)


def _propagate_chunks(core, src, dst, w, t0, t1, o0, o1, sv, dv, wv):
  ebase = core * _E_HALF
  for k in range(_N_CHUNKS):
    off = ebase + k * _CHUNK
    pltpu.sync_copy(src.at[pl.ds(off, _CHUNK)], sv)
    pltpu.sync_copy(dst.at[pl.ds(off, _CHUNK)], dv)
    pltpu.sync_copy(w.at[pl.ds(off, _CHUNK)], wv)
    _edge_loop(t0, t1, o0, o1, sv, dv, wv)


def _sc_phase1_body(ht, src, dst, w, out, t0, t1, o0, o1, sv, dv, wv, bb):
  c = lax.axis_index("c")
  s = lax.axis_index("s")
  d0 = 2 * s
  d1 = d0 + 1
  pltpu.sync_copy(ht.at[d0], t0)
  pltpu.sync_copy(ht.at[d1], t1)
  _zero_columns(o0, o1)
  _propagate_chunks(c, src, dst, w, t0, t1, o0, o1, sv, dv, wv)
  pltpu.sync_copy(o0, out.at[c, d0])
  pltpu.sync_copy(o1, out.at[c, d1])


def _sc_phase2_body(parts, bias, src, dst, w, out,
                    t0, t1, o0, o1, sv, dv, wv, bb):
  c = lax.axis_index("c")
  s = lax.axis_index("s")
  d0 = 2 * s
  d1 = d0 + 1
  pltpu.sync_copy(parts.at[0, d0], t0)
  pltpu.sync_copy(parts.at[1, d0], o0)
  pltpu.sync_copy(parts.at[0, d1], t1)
  pltpu.sync_copy(parts.at[1, d1], o1)
  pltpu.sync_copy(bias.at[d0], bb)
  b0 = bb[...]
  pltpu.sync_copy(bias.at[d1], bb)
  b1 = bb[...]

  def cstep(i, carry):
    sl = pl.ds(i * _LANES, _LANES)
    t0[sl] = jnp.maximum(t0[sl] + o0[sl] + b0, 0.0)
    t1[sl] = jnp.maximum(t1[sl] + o1[sl] + b1, 0.0)
    return carry

  lax.fori_loop(0, _N_VECS, cstep, 0)

  _zero_columns(o0, o1)
  _propagate_chunks(c, src, dst, w, t0, t1, o0, o1, sv, dv, wv)
  pltpu.sync_copy(o0, out.at[c, d0])
  pltpu.sync_copy(o1, out.at[c, d1])


def _make_sc_kernel(body):
  mesh = plsc.VectorSubcoreMesh(core_axis_name="c", subcore_axis_name="s")
  return pl.kernel(
      body,
      out_type=jax.ShapeDtypeStruct((_NUM_CORES, D_HID, N_NODES),
                                    jnp.float32),
      mesh=mesh,
      compiler_params=pltpu.CompilerParams(needs_layout_passes=False),
      scratch_types=[
          pltpu.VMEM((N_NODES,), jnp.float32),
          pltpu.VMEM((N_NODES,), jnp.float32),
          pltpu.VMEM((N_NODES,), jnp.float32),
          pltpu.VMEM((N_NODES,), jnp.float32),
          pltpu.VMEM((_CHUNK,), jnp.int32),
          pltpu.VMEM((_CHUNK,), jnp.int32),
          pltpu.VMEM((_CHUNK,), jnp.float32),
          pltpu.VMEM((_LANES,), jnp.float32),
      ],
  )


def _mm1_body(x_ref, w_ref, out_ref):
  out_ref[...] = lax.dot_general(
      w_ref[...], x_ref[...], (((0,), (1,)), ((), ())),
      preferred_element_type=jnp.float32)


def _final_body(p_ref, w2_ref, b2_ref, out_ref):
  g = p_ref[0] + p_ref[1]
  logits = lax.dot_general(
      g, w2_ref[...], (((0,), (0,)), ((), ())),
      preferred_element_type=jnp.float32)
  z = logits + b2_ref[0][None, :]
  m = jnp.max(z, axis=1, keepdims=True)
  lse = jnp.log(jnp.sum(jnp.exp(z - m), axis=1, keepdims=True)) + m
  out_ref[...] = z - lse


@jax.jit
def kernel(x, edge_index, edge_weight, W1, b1, W2, b2):
  src = edge_index[0].astype(jnp.int32)
  dst = edge_index[1].astype(jnp.int32)
  w = edge_weight.astype(jnp.float32)

  ht = pl.pallas_call(
      _mm1_body,
      out_shape=jax.ShapeDtypeStruct((D_HID, N_NODES), jnp.float32),
  )(x, W1)

  p1 = _make_sc_kernel(_sc_phase1_body)(ht, src, dst, w)

  b1b = jnp.broadcast_to(b1[:, None], (D_HID, _LANES))
  p2 = _make_sc_kernel(_sc_phase2_body)(p1, b1b, src, dst, w)

  out = pl.pallas_call(
      _final_body,
      out_shape=jax.ShapeDtypeStruct((N_NODES, N_CLASSES), jnp.float32),
  )(p2, W2, b2.reshape(1, N_CLASSES))
  return out


# speedup vs baseline: 15.5645x; 2.1061x over previous
import jax
import jax.numpy as jnp
from jax import lax
from jax.experimental import pallas as pl
from jax.experimental.pallas import tpu as pltpu
from jax.experimental.pallas import tpu_sc as plsc

N_NODES = 10000
N_EDGES = 320000
D_IN = 128
D_HID = 32
N_CLASSES = 16

_LANES = 16
_N_EGROUPS = 4
_DIMS = 4
_E_GRP = N_EDGES // _N_EGROUPS
_CHUNK = 4000
_N_CHUNKS = _E_GRP // _CHUNK
_N_VECS = N_NODES // _LANES


def _zero_columns(outs):
  zero = jnp.zeros((_LANES,), jnp.float32)

  @plsc.parallel_loop(0, N_NODES, _LANES)
  def _(i):
    sl = pl.ds(i, _LANES)
    for o in outs:
      o[sl] = zero


def _edge_loop(tabs, outs, sv, dv, wv):
  @plsc.parallel_loop(0, _CHUNK, _LANES, unroll=4)
  def _(i):
    sl = pl.ds(i, _LANES)
    s_idx = sv[sl]
    d_idx = dv[sl]
    w = wv[sl]
    for t in range(_DIMS):
      g = plsc.load_gather(tabs[t], [s_idx]) * w
      plsc.addupdate_scatter(outs[t], [d_idx], g)


def _start_chunk(egrp, k, src, dst, w, buf):
  off = egrp * _E_GRP + k * _CHUNK
  sv, dv, wv, sem = buf
  return (
      pltpu.async_copy(src.at[pl.ds(off, _CHUNK)], sv, sem),
      pltpu.async_copy(dst.at[pl.ds(off, _CHUNK)], dv, sem),
      pltpu.async_copy(w.at[pl.ds(off, _CHUNK)], wv, sem),
  )


def _propagate_chunks(egrp, src, dst, w, tabs, outs, bufs):
  descs = _start_chunk(egrp, 0, src, dst, w, bufs[0])
  for k in range(_N_CHUNKS):
    for d in descs:
      d.wait()
    cur = bufs[k % 2]
    if k + 1 < _N_CHUNKS:
      descs = _start_chunk(egrp, k + 1, src, dst, w, bufs[(k + 1) % 2])
    _edge_loop(tabs, outs, cur[0], cur[1], cur[2])


def _tile_coords():
  c = lax.axis_index("c")
  s = lax.axis_index("s")
  egrp = c * 2 + s // 8
  dbase = (s % 8) * _DIMS
  return egrp, dbase


def _sc_phase1_body(ht, src, dst, w, out, *scratch):
  tabs = scratch[0:4]
  outs = scratch[4:8]
  bufs = (scratch[8:12], scratch[12:16])
  egrp, dbase = _tile_coords()
  for t in range(_DIMS):
    pltpu.sync_copy(ht.at[dbase + t], tabs[t])
  _zero_columns(outs)
  _propagate_chunks(egrp, src, dst, w, tabs, outs, bufs)
  for t in range(_DIMS):
    pltpu.sync_copy(outs[t], out.at[egrp, dbase + t])


def _sc_phase2_body(parts, bias, src, dst, w, out, *scratch):
  tabs = scratch[0:4]
  outs = scratch[4:8]
  bufs = (scratch[8:12], scratch[12:16])
  bb = scratch[16]
  egrp, dbase = _tile_coords()

  for t in range(_DIMS):
    d = dbase + t
    pltpu.sync_copy(parts.at[0, d], tabs[t])
    pltpu.sync_copy(parts.at[1, d], outs[1])
    pltpu.sync_copy(parts.at[2, d], outs[2])
    pltpu.sync_copy(parts.at[3, d], outs[3])
    pltpu.sync_copy(bias.at[d], bb)
    b = bb[...]

    @plsc.parallel_loop(0, N_NODES, _LANES)
    def _(i, t=t, b=b):
      sl = pl.ds(i, _LANES)
      tabs[t][sl] = jnp.maximum(
          tabs[t][sl] + outs[1][sl] + outs[2][sl] + outs[3][sl] + b, 0.0)

  _zero_columns(outs)
  _propagate_chunks(egrp, src, dst, w, tabs, outs, bufs)
  for t in range(_DIMS):
    pltpu.sync_copy(outs[t], out.at[egrp, dbase + t])


def _make_sc_kernel(body):
  mesh = plsc.VectorSubcoreMesh(core_axis_name="c", subcore_axis_name="s")
  return pl.kernel(
      body,
      out_type=jax.ShapeDtypeStruct((_N_EGROUPS, D_HID, N_NODES),
                                    jnp.float32),
      mesh=mesh,
      compiler_params=pltpu.CompilerParams(needs_layout_passes=False),
      scratch_types=[
          *[pltpu.VMEM((N_NODES,), jnp.float32) for _ in range(8)],
          pltpu.VMEM((_CHUNK,), jnp.int32),
          pltpu.VMEM((_CHUNK,), jnp.int32),
          pltpu.VMEM((_CHUNK,), jnp.float32),
          pltpu.SemaphoreType.DMA,
          pltpu.VMEM((_CHUNK,), jnp.int32),
          pltpu.VMEM((_CHUNK,), jnp.int32),
          pltpu.VMEM((_CHUNK,), jnp.float32),
          pltpu.SemaphoreType.DMA,
          pltpu.VMEM((_LANES,), jnp.float32),
      ],
  )


def _mm1_body(x_ref, w_ref, out_ref):
  out_ref[...] = lax.dot_general(
      w_ref[...], x_ref[...], (((0,), (1,)), ((), ())),
      preferred_element_type=jnp.float32)


def _final_body(p_ref, w2_ref, b2_ref, out_ref):
  g = p_ref[0] + p_ref[1] + p_ref[2] + p_ref[3]
  logits = lax.dot_general(
      g, w2_ref[...], (((0,), (0,)), ((), ())),
      preferred_element_type=jnp.float32)
  z = logits + b2_ref[0][None, :]
  m = jnp.max(z, axis=1, keepdims=True)
  lse = jnp.log(jnp.sum(jnp.exp(z - m), axis=1, keepdims=True)) + m
  out_ref[...] = z - lse


@jax.jit
def kernel(x, edge_index, edge_weight, W1, b1, W2, b2):
  src = edge_index[0].astype(jnp.int32)
  dst = edge_index[1].astype(jnp.int32)
  w = edge_weight.astype(jnp.float32)

  ht = pl.pallas_call(
      _mm1_body,
      out_shape=jax.ShapeDtypeStruct((D_HID, N_NODES), jnp.float32),
  )(x, W1)

  p1 = _make_sc_kernel(_sc_phase1_body)(ht, src, dst, w)

  b1b = jnp.broadcast_to(b1[:, None], (D_HID, _LANES))
  p2 = _make_sc_kernel(_sc_phase2_body)(p1, b1b, src, dst, w)

  out = pl.pallas_call(
      _final_body,
      out_shape=jax.ShapeDtypeStruct((N_NODES, N_CLASSES), jnp.float32),
  )(p2, W2, b2.reshape(1, N_CLASSES))
  return out


# speedup vs baseline: 20.5968x; 1.3233x over previous
import jax
import jax.numpy as jnp
from jax import lax
from jax.experimental import pallas as pl
from jax.experimental.pallas import tpu as pltpu
from jax.experimental.pallas import tpu_sc as plsc

N_NODES = 10000
N_EDGES = 320000
D_IN = 128
D_HID = 32
N_CLASSES = 16

_LANES = 16
_CHUNK = 4000
_N_TILES = 32


def _make_sc_propagate(n_feat, dims):
  tiles_per_group = n_feat // dims
  n_groups = _N_TILES // tiles_per_group
  e_grp = N_EDGES // n_groups
  n_chunks = e_grp // _CHUNK
  assert n_chunks * _CHUNK == e_grp

  def zero_columns(outs):
    zero = jnp.zeros((_LANES,), jnp.float32)

    @plsc.parallel_loop(0, N_NODES, _LANES)
    def _(i):
      sl = pl.ds(i, _LANES)
      for o in outs:
        o[sl] = zero

  def edge_loop(tabs, outs, sv, dv, wv):
    @plsc.parallel_loop(0, _CHUNK, _LANES, unroll=4)
    def _(i):
      sl = pl.ds(i, _LANES)
      s_idx = sv[sl]
      d_idx = dv[sl]
      w = wv[sl]
      for t in range(dims):
        g = plsc.load_gather(tabs[t], [s_idx]) * w
        plsc.addupdate_scatter(outs[t], [d_idx], g)

  def start_chunk(egrp, k, src, dst, w, buf):
    off = egrp * e_grp + k * _CHUNK
    sv, dv, wv, sem = buf
    return (
        pltpu.async_copy(src.at[pl.ds(off, _CHUNK)], sv, sem),
        pltpu.async_copy(dst.at[pl.ds(off, _CHUNK)], dv, sem),
        pltpu.async_copy(w.at[pl.ds(off, _CHUNK)], wv, sem),
    )

  def body(ht, src, dst, w, out, *scratch):
    tabs = scratch[0:dims]
    outs = scratch[dims:2 * dims]
    bufs = (scratch[2 * dims:2 * dims + 4], scratch[2 * dims + 4:2 * dims + 8])
    tile = lax.axis_index("c") * 16 + lax.axis_index("s")
    egrp = tile // tiles_per_group
    dbase = (tile % tiles_per_group) * dims
    for t in range(dims):
      pltpu.sync_copy(ht.at[dbase + t], tabs[t])
    zero_columns(outs)
    descs = start_chunk(egrp, 0, src, dst, w, bufs[0])
    for k in range(n_chunks):
      for d in descs:
        d.wait()
      cur = bufs[k % 2]
      if k + 1 < n_chunks:
        descs = start_chunk(egrp, k + 1, src, dst, w, bufs[(k + 1) % 2])
      edge_loop(tabs, outs, cur[0], cur[1], cur[2])
    for t in range(dims):
      pltpu.sync_copy(outs[t], out.at[egrp, dbase + t])

  mesh = plsc.VectorSubcoreMesh(core_axis_name="c", subcore_axis_name="s")
  return pl.kernel(
      body,
      out_type=jax.ShapeDtypeStruct((n_groups, n_feat, N_NODES), jnp.float32),
      mesh=mesh,
      compiler_params=pltpu.CompilerParams(needs_layout_passes=False),
      scratch_types=[
          *[pltpu.VMEM((N_NODES,), jnp.float32) for _ in range(2 * dims)],
          pltpu.VMEM((_CHUNK,), jnp.int32),
          pltpu.VMEM((_CHUNK,), jnp.int32),
          pltpu.VMEM((_CHUNK,), jnp.float32),
          pltpu.SemaphoreType.DMA,
          pltpu.VMEM((_CHUNK,), jnp.int32),
          pltpu.VMEM((_CHUNK,), jnp.int32),
          pltpu.VMEM((_CHUNK,), jnp.float32),
          pltpu.SemaphoreType.DMA,
      ],
  )


def _mm1_body(x_ref, w_ref, out_ref):
  out_ref[...] = lax.dot_general(
      w_ref[...], x_ref[...], (((0,), (1,)), ((), ())),
      preferred_element_type=jnp.float32)


def _mid_body(p_ref, b1_ref, w2_ref, out_ref):
  h = p_ref[0] + p_ref[1] + p_ref[2] + p_ref[3]
  h = jnp.maximum(h + b1_ref[...], 0.0)
  out_ref[...] = lax.dot_general(
      w2_ref[...], h, (((0,), (0,)), ((), ())),
      preferred_element_type=jnp.float32)


def _final_body(p_ref, b2_ref, eye_ref, out_ref):
  g = p_ref[0]
  for i in range(1, p_ref.shape[0]):
    g = g + p_ref[i]
  logits = lax.dot_general(
      g, eye_ref[...], (((0,), (0,)), ((), ())),
      preferred_element_type=jnp.float32)
  z = logits + b2_ref[0][None, :]
  m = jnp.max(z, axis=1, keepdims=True)
  lse = jnp.log(jnp.sum(jnp.exp(z - m), axis=1, keepdims=True)) + m
  out_ref[...] = z - lse


@jax.jit
def kernel(x, edge_index, edge_weight, W1, b1, W2, b2):
  src = edge_index[0].astype(jnp.int32)
  dst = edge_index[1].astype(jnp.int32)
  w = edge_weight.astype(jnp.float32)

  ht = pl.pallas_call(
      _mm1_body,
      out_shape=jax.ShapeDtypeStruct((D_HID, N_NODES), jnp.float32),
  )(x, W1)

  p1 = _make_sc_propagate(D_HID, 4)(ht, src, dst, w)

  gt = pl.pallas_call(
      _mid_body,
      out_shape=jax.ShapeDtypeStruct((N_CLASSES, N_NODES), jnp.float32),
  )(p1, b1.reshape(D_HID, 1), W2)

  p2 = _make_sc_propagate(N_CLASSES, 4)(gt, src, dst, w)

  out = pl.pallas_call(
      _final_body,
      out_shape=jax.ShapeDtypeStruct((N_NODES, N_CLASSES), jnp.float32),
  )(p2, b2.reshape(1, N_CLASSES), jnp.eye(N_CLASSES, dtype=jnp.float32))
  return out
